# Initial kernel scaffold; baseline (speedup 1.0000x reference)
import jax
import jax.numpy as jnp
from jax.experimental import pallas as pl


def kernel(h, edge_index):
    raise NotImplementedError("write your pallas kernel here")


# speedup vs baseline: 5.2742x; 5.2742x over previous
import functools

import jax
import jax.numpy as jnp
from jax import lax
from jax.experimental import pallas as pl
from jax.experimental.pallas import tpu as pltpu
from jax.experimental.pallas import tpu_sc as plsc

N = 10000
D = 128
E = 320000
NC, NS = 2, 16
NW = NC * NS
EPW = E // NW
CHUNK = 128
NCHUNK = -(-EPW // CHUNK)
EPW_PAD = NCHUNK * CHUNK
NPAD = 10240
NROWB = NPAD // 128
RPT = NPAD // NS

_MESH = plsc.VectorSubcoreMesh(core_axis_name="c", subcore_axis_name="s")
_SC_PARAMS = pltpu.CompilerParams(
    needs_layout_passes=False, use_tc_tiling_on_sc=False
)


_GROUPS = NROWB // 8
_GBINS = NPAD // _GROUPS


def _deg_body(src_hbm, dst_hbm, out_hbm, idx_v, hist, rbuf, res,
              hs_sh, hd_sh):
    c = lax.axis_index("c")
    s = lax.axis_index("s")
    wid = s * NC + c

    zero16 = jnp.zeros((16,), jnp.float32)
    ones = jnp.ones((16,), jnp.float32)

    @pl.loop(0, NPAD // 16)
    def _z1(i):
        hist[pl.ds(i * 16, 16)] = zero16

    pltpu.sync_copy(src_hbm.at[wid], idx_v)

    @pl.loop(0, EPW_PAD // 16)
    def _hist_src(i):
        idx = idx_v[pl.ds(i * 16, 16)]
        plsc.addupdate_scatter(hist, [idx], ones)

    pltpu.sync_copy(hist, hs_sh.at[s])

    @pl.loop(0, NPAD // 16)
    def _z2(i):
        hist[pl.ds(i * 16, 16)] = zero16

    pltpu.sync_copy(dst_hbm.at[wid], idx_v)

    @pl.loop(0, EPW_PAD // 16)
    def _hist_dst(i):
        idx = idx_v[pl.ds(i * 16, 16)]
        plsc.addupdate_scatter(hist, [idx], ones)

    pltpu.sync_copy(hist, hd_sh.at[s])

    plsc.subcore_barrier()

    @pl.when(s < _GROUPS)
    def _reduce():
        for oidx, sh in ((0, hs_sh), (1, hd_sh)):
            for r in range(NS):
                pltpu.sync_copy(sh.at[r, pl.ds(s * _GBINS, _GBINS)],
                                rbuf.at[r])

            @pl.loop(0, _GBINS // 16)
            def _sum(k):
                acc = rbuf[0, pl.ds(k * 16, 16)]
                for r in range(1, NS):
                    acc = acc + rbuf[r, pl.ds(k * 16, 16)]
                res[k // 8, pl.ds((k % 8) * 16, 16)] = acc

            pltpu.sync_copy(res, out_hbm.at[c, oidx, pl.ds(s * 8, 8)])


_deg_kernel = pl.kernel(
    _deg_body,
    out_type=jax.ShapeDtypeStruct((NC, 2, NROWB, 128), jnp.float32),
    mesh=_MESH,
    compiler_params=_SC_PARAMS,
    scratch_types=[
        pltpu.VMEM((EPW_PAD,), jnp.int32),
        pltpu.VMEM((NPAD,), jnp.float32),
        pltpu.VMEM((NS, _GBINS), jnp.float32),
        pltpu.VMEM((8, 128), jnp.float32),
        pltpu.VMEM_SHARED((NS, NPAD), jnp.float32),
        pltpu.VMEM_SHARED((NS, NPAD), jnp.float32),
    ],
)


DH = D // 2


def _agg_body(feat_hbm, src_hbm, dst_hbm, out_hbm, sidx, didx, rows, zbuf,
              acc):
    c = lax.axis_index("c")
    s = lax.axis_index("s")
    wid = s * NC + c

    zero16 = jnp.zeros((16,), jnp.float32)

    @pl.loop(0, CHUNK)
    def _zero(r):
        for k in range(DH // 16):
            zbuf[r, pl.ds(k * 16, 16)] = zero16

    pltpu.sync_copy(src_hbm.at[wid], sidx)
    pltpu.sync_copy(dst_hbm.at[wid], didx)

    for half in range(2):
        for b in range(RPT // CHUNK):
            pltpu.sync_copy(zbuf, acc.at[pl.ds(s * RPT + b * CHUNK, CHUNK)])

        plsc.subcore_barrier()

        @pl.loop(0, NCHUNK)
        def _edges(j):
            pltpu.sync_copy(feat_hbm.at[half].at[sidx.at[j]], rows)
            pltpu.sync_copy(rows, acc.at[didx.at[j]], add=True)

        plsc.subcore_barrier()

        pltpu.sync_copy(acc.at[pl.ds(s * RPT, RPT)],
                        out_hbm.at[c, half, pl.ds(s * RPT, RPT)])

        plsc.subcore_barrier()


_agg_kernel = pl.kernel(
    _agg_body,
    out_type=jax.ShapeDtypeStruct((NC, 2, NPAD, DH), jnp.float32),
    mesh=_MESH,
    compiler_params=_SC_PARAMS,
    scratch_types=[
        pltpu.VMEM((NCHUNK, CHUNK), jnp.int32),
        pltpu.VMEM((NCHUNK, CHUNK), jnp.int32),
        pltpu.VMEM((CHUNK, DH), jnp.float32),
        pltpu.VMEM((CHUNK, DH), jnp.float32),
        pltpu.VMEM_SHARED((NPAD, DH), jnp.float32),
    ],
)


def _scale_body(hs0, hs1, hd0, hd1, h_ref, feat_ref, ndst_ref):
    out_deg = hs0[...] + hs1[...]
    norm_src = 1.0 / jnp.maximum(out_deg, 1.0)
    feat_ref[...] = h_ref[...] * norm_src[None]
    in_deg = hd0[...] + hd1[...]
    ndst_ref[...] = 1.0 / jnp.maximum(in_deg, 1.0)


def _comb_body(p00, p01, p10, p11, nd, out_ref):
    lo = (p00[...] + p10[...]) * nd[...]
    hi = (p01[...] + p11[...]) * nd[...]
    out_ref[...] = jnp.concatenate([lo, hi], axis=1)


def kernel(h, edge_index):
    ei = edge_index.astype(jnp.int32)
    src = ei[0].reshape(NW, EPW)
    dst = ei[1].reshape(NW, EPW)
    pad_idx = N + (jnp.arange(EPW_PAD - EPW, dtype=jnp.int32) % (NPAD - N))
    pad_blk = jnp.broadcast_to(pad_idx, (NW, EPW_PAD - EPW))
    src_p = jnp.concatenate([src, pad_blk], axis=1)
    dst_p = jnp.concatenate([dst, pad_blk], axis=1)
    src3 = src_p.reshape(NW, NCHUNK, CHUNK)
    dst3 = dst_p.reshape(NW, NCHUNK, CHUNK)
    h_pad = jnp.pad(h, ((0, NPAD - N), (0, 0)))
    h_t = h_pad.reshape(NPAD, 2, DH).transpose(1, 0, 2)

    hist = _deg_kernel(src_p, dst_p)
    histc = hist.reshape(NC, 2, NPAD, 1)

    feat_t, ndst = pl.pallas_call(
        _scale_body,
        out_shape=(
            jax.ShapeDtypeStruct((2, NPAD, DH), jnp.float32),
            jax.ShapeDtypeStruct((NPAD, 1), jnp.float32),
        ),
    )(histc[0, 0], histc[1, 0], histc[0, 1], histc[1, 1], h_t)

    partial = _agg_kernel(feat_t, src3, dst3)

    out = pl.pallas_call(
        _comb_body,
        out_shape=jax.ShapeDtypeStruct((N, D), jnp.float32),
    )(partial[0, 0, :N], partial[0, 1, :N],
      partial[1, 0, :N], partial[1, 1, :N], ndst[:N])
    return out


# speedup vs baseline: 6.1412x; 1.1644x over previous
import functools

import jax
import jax.numpy as jnp
from jax import lax
from jax.experimental import pallas as pl
from jax.experimental.pallas import tpu as pltpu
from jax.experimental.pallas import tpu_sc as plsc

N = 10000
D = 128
E = 320000
NC, NS = 2, 16
NW = NC * NS
EPW = E // NW
CHUNK = 128
NBUF = 4
NCHUNK = 80
EPW_PAD = NCHUNK * CHUNK
NPAD = 10240
NROWB = NPAD // 128
RPT = NPAD // NS

_MESH = plsc.VectorSubcoreMesh(core_axis_name="c", subcore_axis_name="s")
_SC_PARAMS = pltpu.CompilerParams(
    needs_layout_passes=False, use_tc_tiling_on_sc=False
)


_GROUPS = NROWB // 8
_GBINS = NPAD // _GROUPS


def _deg_body(src_hbm, dst_hbm, out_hbm, idx_v, hist, rbuf, res,
              hs_sh, hd_sh):
    c = lax.axis_index("c")
    s = lax.axis_index("s")
    wid = s * NC + c

    zero16 = jnp.zeros((16,), jnp.float32)
    ones = jnp.ones((16,), jnp.float32)

    @pl.loop(0, NPAD // 16)
    def _z1(i):
        hist[pl.ds(i * 16, 16)] = zero16

    pltpu.sync_copy(src_hbm.at[wid], idx_v)

    @pl.loop(0, EPW_PAD // 16)
    def _hist_src(i):
        idx = idx_v[pl.ds(i * 16, 16)]
        plsc.addupdate_scatter(hist, [idx], ones)

    pltpu.sync_copy(hist, hs_sh.at[s])

    @pl.loop(0, NPAD // 16)
    def _z2(i):
        hist[pl.ds(i * 16, 16)] = zero16

    pltpu.sync_copy(dst_hbm.at[wid], idx_v)

    @pl.loop(0, EPW_PAD // 16)
    def _hist_dst(i):
        idx = idx_v[pl.ds(i * 16, 16)]
        plsc.addupdate_scatter(hist, [idx], ones)

    pltpu.sync_copy(hist, hd_sh.at[s])

    plsc.subcore_barrier()

    @pl.when(s < _GROUPS)
    def _reduce():
        for oidx, sh in ((0, hs_sh), (1, hd_sh)):
            for r in range(NS):
                pltpu.sync_copy(sh.at[r, pl.ds(s * _GBINS, _GBINS)],
                                rbuf.at[r])

            @pl.loop(0, _GBINS // 16)
            def _sum(k):
                acc = rbuf[0, pl.ds(k * 16, 16)]
                for r in range(1, NS):
                    acc = acc + rbuf[r, pl.ds(k * 16, 16)]
                res[k // 8, pl.ds((k % 8) * 16, 16)] = acc

            pltpu.sync_copy(res, out_hbm.at[c, oidx, pl.ds(s * 8, 8)])


_deg_kernel = pl.kernel(
    _deg_body,
    out_type=jax.ShapeDtypeStruct((NC, 2, NROWB, 128), jnp.float32),
    mesh=_MESH,
    compiler_params=_SC_PARAMS,
    scratch_types=[
        pltpu.VMEM((EPW_PAD,), jnp.int32),
        pltpu.VMEM((NPAD,), jnp.float32),
        pltpu.VMEM((NS, _GBINS), jnp.float32),
        pltpu.VMEM((8, 128), jnp.float32),
        pltpu.VMEM_SHARED((NS, NPAD), jnp.float32),
        pltpu.VMEM_SHARED((NS, NPAD), jnp.float32),
    ],
)


DH = D // 2


def _agg_body(feat_hbm, src_hbm, dst_hbm, out_hbm, sidx, didx,
              rows0, rows1, rows2, rows3, zbuf, acc, *sems):
    c = lax.axis_index("c")
    s = lax.axis_index("s")
    wid = s * NC + c
    rows = (rows0, rows1, rows2, rows3)
    gsem = sems[:NBUF]
    ssem = sems[NBUF:]

    zero16 = jnp.zeros((16,), jnp.float32)

    @pl.loop(0, CHUNK)
    def _zero(r):
        for k in range(DH // 16):
            zbuf[r, pl.ds(k * 16, 16)] = zero16

    pltpu.sync_copy(src_hbm.at[wid], sidx)
    pltpu.sync_copy(dst_hbm.at[wid], didx)

    def gather(half, j, b):
        return pltpu.async_copy(
            feat_hbm.at[half].at[sidx.at[j]], rows[b], gsem[b])

    def scatter(j, b):
        return pltpu.async_copy(
            rows[b], acc.at[didx.at[j]], ssem[b], add=True)

    for half in range(2):
        for b in range(RPT // CHUNK):
            pltpu.sync_copy(zbuf, acc.at[pl.ds(s * RPT + b * CHUNK, CHUNK)])

        plsc.subcore_barrier()

        gather(half, 0, 0)

        @pl.loop(0, NCHUNK, step=NBUF)
        def _edges(m):
            for b in range(NBUF):
                k = m + b
                bn = (b + 1) % NBUF
                pltpu.make_async_copy(
                    feat_hbm.at[half].at[sidx.at[k]], rows[b], gsem[b]
                ).wait()
                scatter(k, b)

                @pl.when(k + 1 - NBUF >= 0)
                def _():
                    pltpu.make_async_copy(
                        rows[bn], acc.at[didx.at[k]], ssem[bn]).wait()

                @pl.when(k + 1 < NCHUNK)
                def _():
                    gather(half, k + 1, bn)

        for k in range(NCHUNK - NBUF + 1, NCHUNK):
            b = k % NBUF
            pltpu.make_async_copy(rows[b], acc.at[didx.at[0]], ssem[b]).wait()

        plsc.subcore_barrier()

        pltpu.sync_copy(acc.at[pl.ds(s * RPT, RPT)],
                        out_hbm.at[c, half, pl.ds(s * RPT, RPT)])

        plsc.subcore_barrier()


_agg_kernel = pl.kernel(
    _agg_body,
    out_type=jax.ShapeDtypeStruct((NC, 2, NPAD, DH), jnp.float32),
    mesh=_MESH,
    compiler_params=_SC_PARAMS,
    scratch_types=[
        pltpu.VMEM((NCHUNK, CHUNK), jnp.int32),
        pltpu.VMEM((NCHUNK, CHUNK), jnp.int32),
        pltpu.VMEM((CHUNK, DH), jnp.float32),
        pltpu.VMEM((CHUNK, DH), jnp.float32),
        pltpu.VMEM((CHUNK, DH), jnp.float32),
        pltpu.VMEM((CHUNK, DH), jnp.float32),
        pltpu.VMEM((CHUNK, DH), jnp.float32),
        pltpu.VMEM_SHARED((NPAD, DH), jnp.float32),
    ]
    + [pltpu.SemaphoreType.DMA] * (2 * NBUF),
)


def _scale_body(hs0, hs1, hd0, hd1, h_ref, feat_ref, ndst_ref):
    out_deg = hs0[...] + hs1[...]
    norm_src = 1.0 / jnp.maximum(out_deg, 1.0)
    feat_ref[...] = h_ref[...] * norm_src[None]
    in_deg = hd0[...] + hd1[...]
    ndst_ref[...] = 1.0 / jnp.maximum(in_deg, 1.0)


def _comb_body(p00, p01, p10, p11, nd, out_ref):
    lo = (p00[...] + p10[...]) * nd[...]
    hi = (p01[...] + p11[...]) * nd[...]
    out_ref[...] = jnp.concatenate([lo, hi], axis=1)


def kernel(h, edge_index):
    ei = edge_index.astype(jnp.int32)
    src = ei[0].reshape(NW, EPW)
    dst = ei[1].reshape(NW, EPW)
    pad_idx = N + (jnp.arange(EPW_PAD - EPW, dtype=jnp.int32) % (NPAD - N))
    pad_blk = jnp.broadcast_to(pad_idx, (NW, EPW_PAD - EPW))
    src_p = jnp.concatenate([src, pad_blk], axis=1)
    dst_p = jnp.concatenate([dst, pad_blk], axis=1)
    src3 = src_p.reshape(NW, NCHUNK, CHUNK)
    dst3 = dst_p.reshape(NW, NCHUNK, CHUNK)
    h_pad = jnp.pad(h, ((0, NPAD - N), (0, 0)))
    h_t = h_pad.reshape(NPAD, 2, DH).transpose(1, 0, 2)

    hist = _deg_kernel(src_p, dst_p)
    histc = hist.reshape(NC, 2, NPAD, 1)

    feat_t, ndst = pl.pallas_call(
        _scale_body,
        out_shape=(
            jax.ShapeDtypeStruct((2, NPAD, DH), jnp.float32),
            jax.ShapeDtypeStruct((NPAD, 1), jnp.float32),
        ),
    )(histc[0, 0], histc[1, 0], histc[0, 1], histc[1, 1], h_t)

    partial = _agg_kernel(feat_t, src3, dst3)

    out = pl.pallas_call(
        _comb_body,
        out_shape=jax.ShapeDtypeStruct((N, D), jnp.float32),
    )(partial[0, 0, :N], partial[0, 1, :N],
      partial[1, 0, :N], partial[1, 1, :N], ndst[:N])
    return out
